# Initial kernel scaffold; baseline (speedup 1.0000x reference)
import jax
import jax.numpy as jnp
from jax.experimental import pallas as pl


def kernel(x, lc, edgeCalPara, lcconv_w, lcconv_b, conv2_w, bn1_gamma, bn1_beta, bn2_gamma, bn2_beta):
    raise NotImplementedError("write your pallas kernel here")


# speedup vs baseline: 950.5224x; 950.5224x over previous
import jax
import jax.numpy as jnp
from jax import lax
from jax.experimental import pallas as pl

B, S, N, LC = 4, 128, 256, 5
P = LC * (LC + 1) // 2


def _fused_kernel(x_ref, lc_ref, a_ref, b_ref, lw_ref, lb_ref, w2_ref,
                  g1_ref, b1_ref, g2_ref, b2_ref, out_ref):
    f32 = jnp.float32
    x = x_ref[:]
    a_col = a_ref[:]
    b_col = b_ref[:]
    aw = a_col * lw_ref[:]
    ab = a_col * lb_ref[:] + b_col

    dn = (((1,), (1,)), ((), ()))
    hs = []
    s1 = jnp.zeros((1, N), dtype=f32)
    for b in range(B):
        lcr = lc_ref[b:b + 1, :]
        cio = lax.broadcasted_iota(jnp.int32, (LC, N), 0)
        lcb = jnp.broadcast_to(lcr, (LC, N))
        id1 = jnp.maximum(lcb, cio)
        id2 = jnp.minimum(lcb, cio)
        kic = (id1 * (id1 + 1)) // 2 + id2
        W = jnp.zeros((LC, N), dtype=f32)
        AB = jnp.zeros((LC, N), dtype=f32)
        for p in range(P):
            m = kic == p
            W = jnp.where(m, aw[p:p + 1, :], W)
            AB = jnp.where(m, ab[p:p + 1, :], AB)
        onesN = jnp.ones((N, 1), dtype=f32)
        Cb = lax.dot_general(AB, onesN, (((1,), (0,)), ((), ())),
                             preferred_element_type=f32)
        H = lax.dot_general(x[b], W, dn, precision=lax.Precision.HIGHEST,
                            preferred_element_type=f32)
        O = (cio == lcb).astype(f32)
        CbO = lax.dot_general(Cb, O, (((0,), (0,)), ((), ())),
                              preferred_element_type=f32)
        hb = lax.dot_general(H, O, (((1,), (0,)), ((), ())),
                             precision=lax.Precision.HIGHEST,
                             preferred_element_type=f32) + CbO
        hs.append(hb)
        s1 = s1 + jnp.sum(hb, axis=0, keepdims=True)

    inv_bs = 1.0 / (B * S)
    mean1 = s1 * inv_bs
    ss = jnp.zeros((1, N), dtype=f32)
    for b in range(B):
        d = hs[b] - mean1
        ss = ss + jnp.sum(d * d, axis=0, keepdims=True)
    rstd1 = lax.rsqrt(ss * inv_bs + 1e-5)
    scale1 = rstd1 * g1_ref[:]
    shift1 = b1_ref[:] - mean1 * scale1

    obs = []
    s2 = jnp.zeros((S, 1), dtype=f32)
    w2 = w2_ref[:]
    for b in range(B):
        g = jnp.maximum(hs[b] * scale1 + shift1 + x[b], 0.0)
        ob = lax.dot_general(w2, g, (((1,), (0,)), ((), ())),
                             precision=lax.Precision.HIGHEST,
                             preferred_element_type=f32)
        obs.append(ob)
        s2 = s2 + jnp.sum(ob, axis=1, keepdims=True)

    inv_bn = 1.0 / (B * N)
    mean2 = s2 * inv_bn
    ss2 = jnp.zeros((S, 1), dtype=f32)
    for b in range(B):
        d = obs[b] - mean2
        ss2 = ss2 + jnp.sum(d * d, axis=1, keepdims=True)
    rstd2 = lax.rsqrt(ss2 * inv_bn + 1e-5)
    scale2 = rstd2 * g2_ref[:]
    shift2 = b2_ref[:] - mean2 * scale2
    for b in range(B):
        out_ref[b] = obs[b] * scale2 + shift2


def kernel(x, lc, edgeCalPara, lcconv_w, lcconv_b, conv2_w,
           bn1_gamma, bn1_beta, bn2_gamma, bn2_beta):
    a_col = edgeCalPara[0]
    b_col = edgeCalPara[1]
    out = pl.pallas_call(
        _fused_kernel,
        out_shape=jax.ShapeDtypeStruct((B, S, N), jnp.float32),
    )(x, lc, a_col, b_col, lcconv_w, lcconv_b, conv2_w,
      bn1_gamma.reshape(1, N), bn1_beta.reshape(1, N),
      bn2_gamma.reshape(S, 1), bn2_beta.reshape(S, 1))
    return (out, lc)
